# baseline (speedup 1.0000x reference)
import functools

import jax
import jax.numpy as jnp
from jax.experimental import pallas as pl
from jax.experimental.pallas import tpu as pltpu
from jax.experimental.pallas import tpu_sc as plsc

_EPS = 1e-08
_ENTROPY_WEIGHT = 0.05
_MAX_USAGE_RATIO = 0.4

_T_ROUTER = 512
_BLK = 256
_NC, _NS, _L = 2, 16, 16
_NW = _NC * _NS
_C_DISP = 32
_C_COMB = 16


def _router_body(n, nt_tiles, x_ref, gw_ref, gb_ref, probs_ref, idx_ref,
                 rank_ref, counts_ref, ent_ref, sp_ref, offs_ref, loss_ref):
    i = pl.program_id(0)
    T = x_ref.shape[0]
    E = gw_ref.shape[0]

    @pl.when(i == 0)
    def _():
        counts_ref[...] = jnp.zeros_like(counts_ref)
        ent_ref[...] = jnp.zeros_like(ent_ref)

    x = x_ref[...]
    logits = jax.lax.dot_general(
        gw_ref[...], x, (((1,), (1,)), ((), ())),
        preferred_element_type=jnp.float32,
        precision=jax.lax.Precision.DEFAULT)
    logits = logits + gb_ref[...]
    m = jnp.max(logits, axis=0, keepdims=True)
    ex = jnp.exp(logits - m)
    probs = ex / jnp.sum(ex, axis=0, keepdims=True)
    ent_tile = -jnp.sum(probs * jnp.log(probs + _EPS), axis=(0, 1),
                        keepdims=True)

    rows = jax.lax.broadcasted_iota(jnp.int32, (E, T), 0)
    m1 = jnp.max(probs, axis=0, keepdims=True)
    i1 = jnp.min(jnp.where(probs >= m1, rows, E), axis=0, keepdims=True)
    h1 = rows == i1
    probsm = jnp.where(h1, -jnp.inf, probs)
    m2 = jnp.max(probsm, axis=0, keepdims=True)
    i2 = jnp.min(jnp.where(probsm >= m2, rows, E), axis=0, keepdims=True)
    h2 = rows == i2

    h1f = h1.astype(jnp.float32)
    h2f = h2.astype(jnp.float32)
    hh = h1f + h2f
    r_k = jax.lax.broadcasted_iota(jnp.int32, (T, T), 0)
    c_t = jax.lax.broadcasted_iota(jnp.int32, (T, T), 1)
    triu = (r_k < c_t).astype(jnp.float32)
    c0 = jax.lax.dot_general(
        hh, triu, (((1,), (0,)), ((), ())),
        preferred_element_type=jnp.float32,
        precision=jax.lax.Precision.HIGHEST)
    base = counts_ref[...] + c0
    r1 = jnp.sum(base * h1f, axis=0, keepdims=True)
    r2 = jnp.sum(base * h2f, axis=0, keepdims=True)

    probs_ref[0, 0, 0, :] = m1[0]
    probs_ref[1, 0, 0, :] = m2[0]
    idx_ref[0, 0, 0, :] = i1[0]
    idx_ref[1, 0, 0, :] = i2[0]
    rank_ref[0, 0, 0, :] = r1[0].astype(jnp.int32)
    rank_ref[1, 0, 0, :] = r2[0].astype(jnp.int32)
    new_counts = counts_ref[...] + jnp.sum(hh, axis=1, keepdims=True)
    new_ent = ent_ref[...] + ent_tile
    counts_ref[...] = new_counts
    ent_ref[...] = new_ent

    @pl.when(i == pl.num_programs(0) - 1)
    def _():
        ci = new_counts.astype(jnp.int32)
        padded = ((ci + _BLK - 1) // _BLK) * _BLK
        tril8 = (jax.lax.broadcasted_iota(jnp.int32, (E, E), 0)
                 >= jax.lax.broadcasted_iota(jnp.int32, (E, E), 1))
        csum = jax.lax.dot_general(
            tril8.astype(jnp.float32), padded.astype(jnp.float32),
            (((1,), (0,)), ((), ())),
            preferred_element_type=jnp.float32,
            precision=jax.lax.Precision.HIGHEST)
        offs = csum - padded.astype(jnp.float32)
        eye8 = (jax.lax.broadcasted_iota(jnp.int32, (E, E), 0)
                == jax.lax.broadcasted_iota(jnp.int32, (E, E), 1))
        offs_row = jax.lax.dot_general(
            offs, eye8.astype(jnp.float32), (((0,), (0,)), ((), ())),
            preferred_element_type=jnp.float32,
            precision=jax.lax.Precision.HIGHEST)
        offs_ref[...] = jnp.concatenate(
            [offs_row.astype(jnp.int32),
             jnp.zeros((1, 16 - E), jnp.int32)], axis=1)
        tids = jax.lax.broadcasted_iota(
            jnp.int32, (1, nt_tiles), 1) * _BLK
        te = jnp.sum((csum.astype(jnp.int32) <= tids).astype(jnp.int32),
                     axis=0, keepdims=True)
        te = jnp.minimum(te, E - 1)
        csum_row = jax.lax.dot_general(
            csum, eye8.astype(jnp.float32), (((0,), (0,)), ((), ())),
            preferred_element_type=jnp.float32,
            precision=jax.lax.Precision.HIGHEST)
        nta = (csum_row[:, E - 1:E] / _BLK).astype(jnp.int32)
        sp_ref[...] = jnp.concatenate([te, nta], axis=1)
        ent_loss = _ENTROPY_WEIGHT * (new_ent / n)
        ratios = new_counts / (n + _EPS)
        pen = jnp.sum(jax.nn.relu(ratios - _MAX_USAGE_RATIO),
                      axis=(0, 1), keepdims=True)
        loss_ref[...] = ent_loss + pen


def _run_router(x_flat, gate_w, gate_b2d, nt_tiles, interpret=False):
    n, d = x_flat.shape
    e = gate_w.shape[0]
    nt = n // _T_ROUTER
    out_shape = [
        jax.ShapeDtypeStruct((2, nt, 1, _T_ROUTER), jnp.float32),
        jax.ShapeDtypeStruct((2, nt, 1, _T_ROUTER), jnp.int32),
        jax.ShapeDtypeStruct((2, nt, 1, _T_ROUTER), jnp.int32),
        jax.ShapeDtypeStruct((e, 1), jnp.float32),
        jax.ShapeDtypeStruct((1, 1), jnp.float32),
        jax.ShapeDtypeStruct((1, nt_tiles + 1), jnp.int32),
        jax.ShapeDtypeStruct((1, 16), jnp.int32),
        jax.ShapeDtypeStruct((1, 1), jnp.float32),
    ]
    in_specs = [
        pl.BlockSpec((_T_ROUTER, d), lambda i: (i, 0)),
        pl.BlockSpec((e, d), lambda i: (0, 0)),
        pl.BlockSpec((e, 1), lambda i: (0, 0)),
    ]
    tile3 = pl.BlockSpec((2, 1, 1, _T_ROUTER), lambda i: (0, i, 0, 0))
    out_specs = [
        tile3, tile3, tile3,
        pl.BlockSpec((e, 1), lambda i: (0, 0)),
        pl.BlockSpec((1, 1), lambda i: (0, 0)),
        pl.BlockSpec((1, nt_tiles + 1), lambda i: (0, 0)),
        pl.BlockSpec((1, 16), lambda i: (0, 0)),
        pl.BlockSpec((1, 1), lambda i: (0, 0)),
    ]
    return pl.pallas_call(
        functools.partial(_router_body, n, nt_tiles),
        grid=(nt,), in_specs=in_specs, out_specs=out_specs,
        out_shape=out_shape, interpret=interpret,
    )(x_flat, gate_w, gate_b2d)


def _dispatch_body(n, x_hbm, idx_hbm, rank_hbm, offs_hbm,
                   xs_hbm, dest_hbm,
                   offs_v, idx_v, rank_v, dest2_v, rows0_v, rows1_v,
                   sem_r, sem_w):
    wid = jax.lax.axis_index("s") * _NC + jax.lax.axis_index("c")
    a_per_w = idx_hbm.shape[0] // _NW
    nck = a_per_w // _C_DISP
    base = wid * a_per_w
    tokb = jax.lax.rem(base, n)
    pltpu.sync_copy(offs_hbm, offs_v)
    pltpu.sync_copy(idx_hbm.at[pl.ds(base, a_per_w)], idx_v)
    pltpu.sync_copy(rank_hbm.at[pl.ds(base, a_per_w)], rank_v)
    for c in range(nck):
        for j in range(_C_DISP // _L):
            sl = pl.ds(c * _C_DISP + j * _L, _L)
            dv = plsc.load_gather(offs_v, [idx_v[sl]]) + rank_v[sl]
            dest2_v[c, pl.ds(j * _L, _L)] = dv
    pltpu.sync_copy(dest2_v, dest_hbm.at[wid])
    rows = (rows0_v, rows1_v)
    rd = [None, None]
    wr = [None, None]
    rd[0] = pltpu.async_copy(x_hbm.at[pl.ds(tokb, _C_DISP)], rows0_v, sem_r)
    for c in range(nck):
        k = c % 2
        rd[k].wait()
        wr[k] = pltpu.async_copy(rows[k], xs_hbm.at[dest2_v.at[c]], sem_w)
        if c + 1 < nck:
            kn = (c + 1) % 2
            if wr[kn] is not None:
                wr[kn].wait()
            rd[kn] = pltpu.async_copy(
                x_hbm.at[pl.ds(tokb + (c + 1) * _C_DISP, _C_DISP)],
                rows[kn], sem_r)
    wr[(nck - 2) % 2].wait()
    wr[(nck - 1) % 2].wait()


def _sc_dispatch(x_flat, idx_s, rank_s, offs16, a_pad):
    n, d = x_flat.shape
    a = idx_s.shape[0]
    mesh = plsc.VectorSubcoreMesh(core_axis_name="c", subcore_axis_name="s")
    f = pl.kernel(
        functools.partial(_dispatch_body, n),
        out_type=(jax.ShapeDtypeStruct((a_pad, d), jnp.float32),
                  jax.ShapeDtypeStruct((_NW, a // (_NW * _C_DISP), _C_DISP),
                                       jnp.int32)),
        mesh=mesh,
        compiler_params=pltpu.CompilerParams(needs_layout_passes=False),
        scratch_types=[
            pltpu.VMEM((16,), jnp.int32),
            pltpu.VMEM((a // _NW,), jnp.int32),
            pltpu.VMEM((a // _NW,), jnp.int32),
            pltpu.VMEM((a // (_NW * _C_DISP), _C_DISP), jnp.int32),
            pltpu.VMEM((_C_DISP, d), jnp.float32),
            pltpu.VMEM((_C_DISP, d), jnp.float32),
            pltpu.SemaphoreType.DMA,
            pltpu.SemaphoreType.DMA,
        ],
    )
    xs, dest3 = f(x_flat, idx_s, rank_s, offs16)
    return xs, dest3.reshape(a)


def _gmm_body(sp_ref, xs_ref, w_ref, b_ref, out_ref):
    i = pl.program_id(0)
    nt = sp_ref.shape[0] - 1
    nt_act = sp_ref[nt]

    @pl.when(i < nt_act)
    def _():
        xb = xs_ref[...].astype(jnp.bfloat16)
        wb = w_ref[0].astype(jnp.bfloat16)
        y = jax.lax.dot_general(xb, wb, (((1,), (1,)), ((), ())),
                                preferred_element_type=jnp.float32)
        out_ref[...] = y + b_ref[0]


def _tc_gmm(sp, xs, expert_w, expert_b3, interpret=False):
    a_pad, d = xs.shape
    e, h, _ = expert_w.shape
    nt = a_pad // _BLK
    grid_spec = pltpu.PrefetchScalarGridSpec(
        num_scalar_prefetch=1,
        grid=(nt,),
        in_specs=[
            pl.BlockSpec((_BLK, d),
                         lambda i, spr: (jnp.where(i < spr[nt], i, 0), 0)),
            pl.BlockSpec((1, h, d), lambda i, spr: (spr[i], 0, 0)),
            pl.BlockSpec((1, 1, h), lambda i, spr: (spr[i], 0, 0)),
        ],
        out_specs=pl.BlockSpec(
            (_BLK, h), lambda i, spr: (jnp.where(i < spr[nt], i, nt), 0)),
    )
    return pl.pallas_call(
        _gmm_body, grid_spec=grid_spec,
        out_shape=jax.ShapeDtypeStruct((a_pad + _BLK, h), jnp.float32),
        interpret=interpret,
    )(sp, xs, expert_w, expert_b3)


def _combine_body(n, ys_hbm, dest_hbm, probs_hbm, out_hbm,
                  d0_v, d1_v, p0_v, p1_v, r0a, r0b, r1a, r1b, sem_g, sem_w):
    wid = jax.lax.axis_index("s") * _NC + jax.lax.axis_index("c")
    h = ys_hbm.shape[1]
    t_per_w = n // _NW
    nck = t_per_w // _C_COMB
    t0 = wid * t_per_w
    pltpu.sync_copy(dest_hbm.at[pl.ds(t0, t_per_w)], d0_v)
    pltpu.sync_copy(dest_hbm.at[pl.ds(n + t0, t_per_w)], d1_v)
    pltpu.sync_copy(probs_hbm.at[pl.ds(t0, t_per_w)], p0_v)
    pltpu.sync_copy(probs_hbm.at[pl.ds(n + t0, t_per_w)], p1_v)
    R0 = (r0a, r0b)
    R1 = (r1a, r1b)

    def gath(c, k):
        sl = pl.ds(c * _C_COMB, _C_COMB)
        return (pltpu.async_copy(ys_hbm.at[d0_v.at[sl]], R0[k], sem_g),
                pltpu.async_copy(ys_hbm.at[d1_v.at[sl]], R1[k], sem_g))

    wb = [None, None]
    pend = {0: gath(0, 0)}
    for c in range(nck):
        k = c % 2
        for dsc in pend.pop(c):
            dsc.wait()
        if c + 1 < nck:
            kn = (c + 1) % 2
            if wb[kn] is not None:
                wb[kn].wait()
                wb[kn] = None
            pend[c + 1] = gath(c + 1, kn)

        def body(t, carry):
            tsel = jnp.full((_L,), t, jnp.int32) + c * _C_COMB
            p0s = plsc.load_gather(p0_v, [tsel])
            p1s = plsc.load_gather(p1_v, [tsel])
            for hh in range(h // _L):
                sl = pl.ds(hh * _L, _L)
                R0[k][t, sl] = p0s * R0[k][t, sl] + p1s * R1[k][t, sl]
            return carry

        jax.lax.fori_loop(0, _C_COMB, body, 0)
        wb[k] = pltpu.async_copy(
            R0[k], out_hbm.at[pl.ds(t0 + c * _C_COMB, _C_COMB)], sem_w)
    for k in range(2):
        if wb[k] is not None:
            wb[k].wait()


def _sc_combine(ys, dest, probs_s):
    a_pad, h = ys.shape
    n = probs_s.shape[0] // 2
    mesh = plsc.VectorSubcoreMesh(core_axis_name="c", subcore_axis_name="s")
    f = pl.kernel(
        functools.partial(_combine_body, n),
        out_type=jax.ShapeDtypeStruct((n, h), jnp.float32),
        mesh=mesh,
        compiler_params=pltpu.CompilerParams(needs_layout_passes=False),
        scratch_types=[
            pltpu.VMEM((n // _NW,), jnp.int32),
            pltpu.VMEM((n // _NW,), jnp.int32),
            pltpu.VMEM((n // _NW,), jnp.float32),
            pltpu.VMEM((n // _NW,), jnp.float32),
            pltpu.VMEM((_C_COMB, h), jnp.float32),
            pltpu.VMEM((_C_COMB, h), jnp.float32),
            pltpu.VMEM((_C_COMB, h), jnp.float32),
            pltpu.VMEM((_C_COMB, h), jnp.float32),
            pltpu.SemaphoreType.DMA,
            pltpu.SemaphoreType.DMA,
        ],
    )
    return f(ys, dest, probs_s)


def kernel(x, gate_w, gate_b, expert_w, expert_b):
    b, s, d = x.shape
    n = b * s
    e, h, _ = expert_w.shape
    a = 2 * n
    nt = (a + e * _BLK) // _BLK
    a_pad = nt * _BLK
    x_flat = x.reshape(n, d)

    probs, idx, rank, counts, ent, sp2, offs2, loss2 = _run_router(
        x_flat, gate_w, gate_b.reshape(-1, 1), nt)
    sp = sp2.reshape(nt + 1)
    offs16 = offs2.reshape(16)

    idx_s = idx.reshape(a)
    rank_s = rank.reshape(a)
    probs_s = probs.reshape(a)

    xs, dest = _sc_dispatch(x_flat, idx_s, rank_s, offs16, a_pad)
    ys = _tc_gmm(sp, xs, expert_w, expert_b.reshape(e, 1, h))
    out = _sc_combine(ys, dest, probs_s)

    loss = loss2[0, 0]
    return out.reshape(b, s, -1), loss


# speedup vs baseline: 1.0748x; 1.0748x over previous
import functools

import jax
import jax.numpy as jnp
from jax.experimental import pallas as pl
from jax.experimental.pallas import tpu as pltpu
from jax.experimental.pallas import tpu_sc as plsc

_EPS = 1e-08
_ENTROPY_WEIGHT = 0.05
_MAX_USAGE_RATIO = 0.4

_T_ROUTER = 512
_BLK = 512
_NC, _NS, _L = 2, 16, 16
_NW = _NC * _NS
_C_DISP = 32
_C_COMB = 16


def _router_body(n, nt_tiles, x_ref, gw_ref, gb_ref, probs_ref, idx_ref,
                 rank_ref, counts_ref, ent_ref, sp_ref, offs_ref, loss_ref):
    i = pl.program_id(0)
    T = x_ref.shape[0]
    E = gw_ref.shape[0]

    @pl.when(i == 0)
    def _():
        counts_ref[...] = jnp.zeros_like(counts_ref)
        ent_ref[...] = jnp.zeros_like(ent_ref)

    x = x_ref[...]
    logits = jax.lax.dot_general(
        gw_ref[...], x, (((1,), (1,)), ((), ())),
        preferred_element_type=jnp.float32,
        precision=jax.lax.Precision.DEFAULT)
    logits = logits + gb_ref[...]
    m = jnp.max(logits, axis=0, keepdims=True)
    ex = jnp.exp(logits - m)
    probs = ex / jnp.sum(ex, axis=0, keepdims=True)
    ent_tile = -jnp.sum(probs * jnp.log(probs + _EPS), axis=(0, 1),
                        keepdims=True)

    rows = jax.lax.broadcasted_iota(jnp.int32, (E, T), 0)
    m1 = jnp.max(probs, axis=0, keepdims=True)
    i1 = jnp.min(jnp.where(probs >= m1, rows, E), axis=0, keepdims=True)
    h1 = rows == i1
    probsm = jnp.where(h1, -jnp.inf, probs)
    m2 = jnp.max(probsm, axis=0, keepdims=True)
    i2 = jnp.min(jnp.where(probsm >= m2, rows, E), axis=0, keepdims=True)
    h2 = rows == i2

    h1f = h1.astype(jnp.float32)
    h2f = h2.astype(jnp.float32)
    hh = h1f + h2f
    r_k = jax.lax.broadcasted_iota(jnp.int32, (T, T), 0)
    c_t = jax.lax.broadcasted_iota(jnp.int32, (T, T), 1)
    triu = (r_k < c_t).astype(jnp.float32)
    c0 = jax.lax.dot_general(
        hh, triu, (((1,), (0,)), ((), ())),
        preferred_element_type=jnp.float32,
        precision=jax.lax.Precision.HIGHEST)
    base = counts_ref[...] + c0
    r1 = jnp.sum(base * h1f, axis=0, keepdims=True)
    r2 = jnp.sum(base * h2f, axis=0, keepdims=True)

    probs_ref[0, 0, 0, :] = m1[0]
    probs_ref[1, 0, 0, :] = m2[0]
    idx_ref[0, 0, 0, :] = i1[0]
    idx_ref[1, 0, 0, :] = i2[0]
    rank_ref[0, 0, 0, :] = r1[0].astype(jnp.int32)
    rank_ref[1, 0, 0, :] = r2[0].astype(jnp.int32)
    new_counts = counts_ref[...] + jnp.sum(hh, axis=1, keepdims=True)
    new_ent = ent_ref[...] + ent_tile
    counts_ref[...] = new_counts
    ent_ref[...] = new_ent

    @pl.when(i == pl.num_programs(0) - 1)
    def _():
        ci = new_counts.astype(jnp.int32)
        padded = ((ci + _BLK - 1) // _BLK) * _BLK
        tril8 = (jax.lax.broadcasted_iota(jnp.int32, (E, E), 0)
                 >= jax.lax.broadcasted_iota(jnp.int32, (E, E), 1))
        csum = jax.lax.dot_general(
            tril8.astype(jnp.float32), padded.astype(jnp.float32),
            (((1,), (0,)), ((), ())),
            preferred_element_type=jnp.float32,
            precision=jax.lax.Precision.HIGHEST)
        offs = csum - padded.astype(jnp.float32)
        eye8 = (jax.lax.broadcasted_iota(jnp.int32, (E, E), 0)
                == jax.lax.broadcasted_iota(jnp.int32, (E, E), 1))
        offs_row = jax.lax.dot_general(
            offs, eye8.astype(jnp.float32), (((0,), (0,)), ((), ())),
            preferred_element_type=jnp.float32,
            precision=jax.lax.Precision.HIGHEST)
        offs_ref[...] = jnp.concatenate(
            [offs_row.astype(jnp.int32),
             jnp.zeros((1, 16 - E), jnp.int32)], axis=1)
        tids = jax.lax.broadcasted_iota(
            jnp.int32, (1, nt_tiles), 1) * _BLK
        te = jnp.sum((csum.astype(jnp.int32) <= tids).astype(jnp.int32),
                     axis=0, keepdims=True)
        te = jnp.minimum(te, E - 1)
        csum_row = jax.lax.dot_general(
            csum, eye8.astype(jnp.float32), (((0,), (0,)), ((), ())),
            preferred_element_type=jnp.float32,
            precision=jax.lax.Precision.HIGHEST)
        nta = (csum_row[:, E - 1:E] / _BLK).astype(jnp.int32)
        sp_ref[...] = jnp.concatenate([te, nta], axis=1)
        ent_loss = _ENTROPY_WEIGHT * (new_ent / n)
        ratios = new_counts / (n + _EPS)
        pen = jnp.sum(jax.nn.relu(ratios - _MAX_USAGE_RATIO),
                      axis=(0, 1), keepdims=True)
        loss_ref[...] = ent_loss + pen


def _run_router(x_flat, gate_w, gate_b2d, nt_tiles, interpret=False):
    n, d = x_flat.shape
    e = gate_w.shape[0]
    nt = n // _T_ROUTER
    out_shape = [
        jax.ShapeDtypeStruct((2, nt, 1, _T_ROUTER), jnp.float32),
        jax.ShapeDtypeStruct((2, nt, 1, _T_ROUTER), jnp.int32),
        jax.ShapeDtypeStruct((2, nt, 1, _T_ROUTER), jnp.int32),
        jax.ShapeDtypeStruct((e, 1), jnp.float32),
        jax.ShapeDtypeStruct((1, 1), jnp.float32),
        jax.ShapeDtypeStruct((1, nt_tiles + 1), jnp.int32),
        jax.ShapeDtypeStruct((1, 16), jnp.int32),
        jax.ShapeDtypeStruct((1, 1), jnp.float32),
    ]
    in_specs = [
        pl.BlockSpec((_T_ROUTER, d), lambda i: (i, 0)),
        pl.BlockSpec((e, d), lambda i: (0, 0)),
        pl.BlockSpec((e, 1), lambda i: (0, 0)),
    ]
    tile3 = pl.BlockSpec((2, 1, 1, _T_ROUTER), lambda i: (0, i, 0, 0))
    out_specs = [
        tile3, tile3, tile3,
        pl.BlockSpec((e, 1), lambda i: (0, 0)),
        pl.BlockSpec((1, 1), lambda i: (0, 0)),
        pl.BlockSpec((1, nt_tiles + 1), lambda i: (0, 0)),
        pl.BlockSpec((1, 16), lambda i: (0, 0)),
        pl.BlockSpec((1, 1), lambda i: (0, 0)),
    ]
    return pl.pallas_call(
        functools.partial(_router_body, n, nt_tiles),
        grid=(nt,), in_specs=in_specs, out_specs=out_specs,
        out_shape=out_shape, interpret=interpret,
    )(x_flat, gate_w, gate_b2d)


def _dispatch_body(n, x_hbm, idx_hbm, rank_hbm, offs_hbm,
                   xs_hbm, dest_hbm,
                   offs_v, idx_v, rank_v, dest2_v, rows0_v, rows1_v,
                   rows2_v, sem_r, sem_w):
    wid = jax.lax.axis_index("s") * _NC + jax.lax.axis_index("c")
    a_per_w = idx_hbm.shape[0] // _NW
    nck = a_per_w // _C_DISP
    base = wid * a_per_w
    tokb = jax.lax.rem(base, n)
    pltpu.sync_copy(offs_hbm, offs_v)
    pltpu.sync_copy(idx_hbm.at[pl.ds(base, a_per_w)], idx_v)
    pltpu.sync_copy(rank_hbm.at[pl.ds(base, a_per_w)], rank_v)
    for c in range(nck):
        for j in range(_C_DISP // _L):
            sl = pl.ds(c * _C_DISP + j * _L, _L)
            dv = plsc.load_gather(offs_v, [idx_v[sl]]) + rank_v[sl]
            dest2_v[c, pl.ds(j * _L, _L)] = dv
    pltpu.sync_copy(dest2_v, dest_hbm.at[wid])
    rows = (rows0_v, rows1_v, rows2_v)
    rd = [None, None, None]
    wr = [None, None, None]
    rd[0] = pltpu.async_copy(x_hbm.at[pl.ds(tokb, _C_DISP)], rows0_v, sem_r)
    rd[1] = pltpu.async_copy(
        x_hbm.at[pl.ds(tokb + _C_DISP, _C_DISP)], rows1_v, sem_r)
    for c in range(nck):
        k = c % 3
        rd[k].wait()
        wr[k] = pltpu.async_copy(rows[k], xs_hbm.at[dest2_v.at[c]], sem_w)
        if c + 2 < nck:
            kn = (c + 2) % 3
            if wr[kn] is not None:
                wr[kn].wait()
            rd[kn] = pltpu.async_copy(
                x_hbm.at[pl.ds(tokb + (c + 2) * _C_DISP, _C_DISP)],
                rows[kn], sem_r)
    for k in range(3):
        if wr[k] is not None:
            wr[k].wait()


def _sc_dispatch(x_flat, idx_s, rank_s, offs16, a_pad):
    n, d = x_flat.shape
    a = idx_s.shape[0]
    mesh = plsc.VectorSubcoreMesh(core_axis_name="c", subcore_axis_name="s")
    f = pl.kernel(
        functools.partial(_dispatch_body, n),
        out_type=(jax.ShapeDtypeStruct((a_pad, d), jnp.float32),
                  jax.ShapeDtypeStruct((_NW, a // (_NW * _C_DISP), _C_DISP),
                                       jnp.int32)),
        mesh=mesh,
        compiler_params=pltpu.CompilerParams(needs_layout_passes=False),
        scratch_types=[
            pltpu.VMEM((16,), jnp.int32),
            pltpu.VMEM((a // _NW,), jnp.int32),
            pltpu.VMEM((a // _NW,), jnp.int32),
            pltpu.VMEM((a // (_NW * _C_DISP), _C_DISP), jnp.int32),
            pltpu.VMEM((_C_DISP, d), jnp.float32),
            pltpu.VMEM((_C_DISP, d), jnp.float32),
            pltpu.VMEM((_C_DISP, d), jnp.float32),
            pltpu.SemaphoreType.DMA,
            pltpu.SemaphoreType.DMA,
        ],
    )
    xs, dest3 = f(x_flat, idx_s, rank_s, offs16)
    return xs, dest3.reshape(a)


def _gmm_body(sp_ref, xs_ref, w_ref, b_ref, out_ref):
    i = pl.program_id(0)
    nt = sp_ref.shape[0] - 1
    nt_act = sp_ref[nt]

    @pl.when(i < nt_act)
    def _():
        xb = xs_ref[...].astype(jnp.bfloat16)
        wb = w_ref[0].astype(jnp.bfloat16)
        y = jax.lax.dot_general(xb, wb, (((1,), (1,)), ((), ())),
                                preferred_element_type=jnp.float32)
        out_ref[...] = y + b_ref[0]


def _tc_gmm(sp, xs, expert_w, expert_b3, interpret=False):
    a_pad, d = xs.shape
    e, h, _ = expert_w.shape
    nt = a_pad // _BLK
    grid_spec = pltpu.PrefetchScalarGridSpec(
        num_scalar_prefetch=1,
        grid=(nt,),
        in_specs=[
            pl.BlockSpec((_BLK, d),
                         lambda i, spr: (jnp.where(i < spr[nt], i, 0), 0)),
            pl.BlockSpec((1, h, d), lambda i, spr: (spr[i], 0, 0)),
            pl.BlockSpec((1, 1, h), lambda i, spr: (spr[i], 0, 0)),
        ],
        out_specs=pl.BlockSpec(
            (_BLK, h), lambda i, spr: (jnp.where(i < spr[nt], i, nt), 0)),
    )
    return pl.pallas_call(
        _gmm_body, grid_spec=grid_spec,
        out_shape=jax.ShapeDtypeStruct((a_pad + _BLK, h), jnp.float32),
        interpret=interpret,
    )(sp, xs, expert_w, expert_b3)


def _combine_body(n, ys_hbm, dest_hbm, probs_hbm, out_hbm,
                  d0_v, d1_v, p0_v, p1_v, r0a, r0b, r0c, r1a, r1b, r1c,
                  sem_g, sem_w):
    wid = jax.lax.axis_index("s") * _NC + jax.lax.axis_index("c")
    h = ys_hbm.shape[1]
    t_per_w = n // _NW
    nck = t_per_w // _C_COMB
    t0 = wid * t_per_w
    pltpu.sync_copy(dest_hbm.at[pl.ds(t0, t_per_w)], d0_v)
    pltpu.sync_copy(dest_hbm.at[pl.ds(n + t0, t_per_w)], d1_v)
    pltpu.sync_copy(probs_hbm.at[pl.ds(t0, t_per_w)], p0_v)
    pltpu.sync_copy(probs_hbm.at[pl.ds(n + t0, t_per_w)], p1_v)
    R0 = (r0a, r0b, r0c)
    R1 = (r1a, r1b, r1c)

    def gath(c, k):
        sl = pl.ds(c * _C_COMB, _C_COMB)
        return (pltpu.async_copy(ys_hbm.at[d0_v.at[sl]], R0[k], sem_g),
                pltpu.async_copy(ys_hbm.at[d1_v.at[sl]], R1[k], sem_g))

    wb = [None, None, None]
    pend = {0: gath(0, 0), 1: gath(1, 1)}
    for c in range(nck):
        k = c % 3
        for dsc in pend.pop(c):
            dsc.wait()
        if c + 2 < nck:
            kn = (c + 2) % 3
            if wb[kn] is not None:
                wb[kn].wait()
                wb[kn] = None
            pend[c + 2] = gath(c + 2, kn)

        def body(t, carry):
            tsel = jnp.full((_L,), t, jnp.int32) + c * _C_COMB
            p0s = plsc.load_gather(p0_v, [tsel])
            p1s = plsc.load_gather(p1_v, [tsel])
            for hh in range(h // _L):
                sl = pl.ds(hh * _L, _L)
                R0[k][t, sl] = p0s * R0[k][t, sl] + p1s * R1[k][t, sl]
            return carry

        jax.lax.fori_loop(0, _C_COMB, body, 0)
        wb[k] = pltpu.async_copy(
            R0[k], out_hbm.at[pl.ds(t0 + c * _C_COMB, _C_COMB)], sem_w)
    for k in range(3):
        if wb[k] is not None:
            wb[k].wait()


def _sc_combine(ys, dest, probs_s):
    a_pad, h = ys.shape
    n = probs_s.shape[0] // 2
    mesh = plsc.VectorSubcoreMesh(core_axis_name="c", subcore_axis_name="s")
    f = pl.kernel(
        functools.partial(_combine_body, n),
        out_type=jax.ShapeDtypeStruct((n, h), jnp.float32),
        mesh=mesh,
        compiler_params=pltpu.CompilerParams(needs_layout_passes=False),
        scratch_types=[
            pltpu.VMEM((n // _NW,), jnp.int32),
            pltpu.VMEM((n // _NW,), jnp.int32),
            pltpu.VMEM((n // _NW,), jnp.float32),
            pltpu.VMEM((n // _NW,), jnp.float32),
            pltpu.VMEM((_C_COMB, h), jnp.float32),
            pltpu.VMEM((_C_COMB, h), jnp.float32),
            pltpu.VMEM((_C_COMB, h), jnp.float32),
            pltpu.VMEM((_C_COMB, h), jnp.float32),
            pltpu.VMEM((_C_COMB, h), jnp.float32),
            pltpu.VMEM((_C_COMB, h), jnp.float32),
            pltpu.SemaphoreType.DMA,
            pltpu.SemaphoreType.DMA,
        ],
    )
    return f(ys, dest, probs_s)


def kernel(x, gate_w, gate_b, expert_w, expert_b):
    b, s, d = x.shape
    n = b * s
    e, h, _ = expert_w.shape
    a = 2 * n
    nt = (a + e * _BLK) // _BLK
    a_pad = nt * _BLK
    x_flat = x.reshape(n, d)

    probs, idx, rank, counts, ent, sp2, offs2, loss2 = _run_router(
        x_flat, gate_w, gate_b.reshape(-1, 1), nt)
    sp = sp2.reshape(nt + 1)
    offs16 = offs2.reshape(16)

    idx_s = idx.reshape(a)
    rank_s = rank.reshape(a)
    probs_s = probs.reshape(a)

    xs, dest = _sc_dispatch(x_flat, idx_s, rank_s, offs16, a_pad)
    ys = _tc_gmm(sp, xs, expert_w, expert_b.reshape(e, 1, h))
    out = _sc_combine(ys, dest, probs_s)

    loss = loss2[0, 0]
    return out.reshape(b, s, -1), loss
